# Initial kernel scaffold; baseline (speedup 1.0000x reference)
import jax
import jax.numpy as jnp
from jax.experimental import pallas as pl


def kernel(feat, W_rel0, b_rel0, W_root0, gn_w0, gn_b0, gn_ms0, W_rel1, b_rel1, W_root1, gn_w1, gn_b1, gn_ms1, W_rel2, b_rel2, W_root2, gn_w2, gn_b2, gn_ms2, W_rel3, b_rel3, W_root3, gn_w3, gn_b3, gn_ms3, W_lin, b_lin):
    raise NotImplementedError("write your pallas kernel here")


# speedup vs baseline: 13.8545x; 13.8545x over previous
import jax
import jax.numpy as jnp
from jax.experimental import pallas as pl
from jax.experimental.pallas import tpu as pltpu

_N = 100000
_D = 128
_C = 10
_BLK = 2000
_NB = _N // _BLK
_EPS = 1e-5
_SLOPE = 0.1


def _conv_block(b, x, w2, brel, carry_ref):
    p = jnp.dot(x, w2, preferred_element_type=jnp.float32)
    a = p[:, :_D]
    bb = p[:, _D:]

    @pl.when(b == 0)
    def _():
        carry_ref[...] = jnp.zeros_like(carry_ref)

    prev = carry_ref[...]
    shifted = jnp.concatenate([prev, a[:-1, :]], axis=0)
    carry_ref[...] = a[-1:, :]
    return shifted + bb + brel


def _stats_update(b, conv, stats_ref):
    st = jnp.concatenate(
        [jnp.sum(conv, axis=0, keepdims=True),
         jnp.sum(conv * conv, axis=0, keepdims=True)], axis=0)

    @pl.when(b == 0)
    def _():
        stats_ref[...] = jnp.zeros_like(stats_ref)

    stats_ref[...] += st


def _normalize(conv_in, stats, gnw, gnb, gnms):
    mean = stats[0:1, :] * (1.0 / _N)
    ex2 = stats[1:2, :] * (1.0 / _N)
    var = ex2 - gnms * (2.0 - gnms) * mean * mean
    inv = jax.lax.rsqrt(var + _EPS)
    return gnw * (conv_in - gnms * mean) * inv + gnb


def _pass0_kernel(x_ref, w2_ref, brel_ref, conv_ref, stats_ref, carry_ref):
    b = pl.program_id(0)
    conv = _conv_block(b, x_ref[...], w2_ref[...], brel_ref[...], carry_ref)
    conv_ref[...] = conv
    _stats_update(b, conv, stats_ref)


def _mid_kernel(cin_ref, sin_ref, gnw_ref, gnb_ref, gnms_ref, w2_ref, brel_ref,
                conv_ref, stats_ref, carry_ref):
    b = pl.program_id(0)
    y = _normalize(cin_ref[...], sin_ref[...], gnw_ref[...], gnb_ref[...],
                   gnms_ref[...])
    x = jnp.where(y >= 0, y, _SLOPE * y)
    conv = _conv_block(b, x, w2_ref[...], brel_ref[...], carry_ref)
    conv_ref[...] = conv
    _stats_update(b, conv, stats_ref)


def _final_kernel(cin_ref, sin_ref, gnw_ref, gnb_ref, gnms_ref, x0_ref,
                  wlt_ref, blin_ref, out_ref, acc_ref):
    b = pl.program_id(0)
    y = _normalize(cin_ref[...], sin_ref[...], gnw_ref[...], gnb_ref[...],
                   gnms_ref[...])
    z = jnp.maximum(x0_ref[...] + y, 0.0)

    @pl.when(b == 0)
    def _():
        acc_ref[...] = jnp.zeros_like(acc_ref)

    acc_ref[...] += jnp.sum(z, axis=0, keepdims=True)

    @pl.when(b == _NB - 1)
    def _():
        pooled = acc_ref[...] * (1.0 / _N)
        logits = jnp.dot(pooled, wlt_ref[...],
                         preferred_element_type=jnp.float32) + blin_ref[...]
        out_ref[...] = jax.nn.softplus(logits)


def _row_spec():
    return pl.BlockSpec((_BLK, _D), lambda b: (b, 0))


def _const_spec(shape):
    return pl.BlockSpec(shape, lambda b, _s=shape: (0,) * len(_s))


def kernel(feat, W_rel0, b_rel0, W_root0, gn_w0, gn_b0, gn_ms0,
           W_rel1, b_rel1, W_root1, gn_w1, gn_b1, gn_ms1,
           W_rel2, b_rel2, W_root2, gn_w2, gn_b2, gn_ms2,
           W_rel3, b_rel3, W_root3, gn_w3, gn_b3, gn_ms3,
           W_lin, b_lin):
    x0 = feat[0]
    f32 = jnp.float32

    def w2(Wr, Wo):
        return jnp.concatenate([Wr.T, Wo.T], axis=1)

    row = _row_spec()
    conv_shape = jax.ShapeDtypeStruct((_N, _D), f32)
    stats_shape = jax.ShapeDtypeStruct((2, _D), f32)

    conv, stats = pl.pallas_call(
        _pass0_kernel,
        grid=(_NB,),
        in_specs=[row, _const_spec((_D, 2 * _D)), _const_spec((1, _D))],
        out_specs=[row, _const_spec((2, _D))],
        out_shape=[conv_shape, stats_shape],
        scratch_shapes=[pltpu.VMEM((1, _D), f32)],
    )(x0, w2(W_rel0, W_root0), b_rel0.reshape(1, _D))

    layers = [
        (gn_w0, gn_b0, gn_ms0, W_rel1, W_root1, b_rel1),
        (gn_w1, gn_b1, gn_ms1, W_rel2, W_root2, b_rel2),
        (gn_w2, gn_b2, gn_ms2, W_rel3, W_root3, b_rel3),
    ]
    for gw, gb, gms, Wr, Wo, br in layers:
        conv, stats = pl.pallas_call(
            _mid_kernel,
            grid=(_NB,),
            in_specs=[row, _const_spec((2, _D)), _const_spec((1, _D)),
                      _const_spec((1, _D)), _const_spec((1, _D)),
                      _const_spec((_D, 2 * _D)), _const_spec((1, _D))],
            out_specs=[row, _const_spec((2, _D))],
            out_shape=[conv_shape, stats_shape],
            scratch_shapes=[pltpu.VMEM((1, _D), f32)],
        )(conv, stats, gw.reshape(1, _D), gb.reshape(1, _D),
          gms.reshape(1, _D), w2(Wr, Wo), br.reshape(1, _D))

    out = pl.pallas_call(
        _final_kernel,
        grid=(_NB,),
        in_specs=[row, _const_spec((2, _D)), _const_spec((1, _D)),
                  _const_spec((1, _D)), _const_spec((1, _D)), row,
                  _const_spec((_D, _C)), _const_spec((1, _C))],
        out_specs=_const_spec((1, _C)),
        out_shape=jax.ShapeDtypeStruct((1, _C), f32),
        scratch_shapes=[pltpu.VMEM((1, _D), f32)],
    )(conv, stats, gn_w3.reshape(1, _D), gn_b3.reshape(1, _D),
      gn_ms3.reshape(1, _D), x0, W_lin.T, b_lin.reshape(1, _C))

    return out.reshape(_C)


# speedup vs baseline: 14.5259x; 1.0485x over previous
import jax
import jax.numpy as jnp
from jax.experimental import pallas as pl
from jax.experimental.pallas import tpu as pltpu

_N = 100000
_D = 128
_C = 10
_BLK = 2000
_NB = _N // _BLK
_EPS = 1e-5
_SLOPE = 0.1


def _fused_kernel(x0_ref, w2_ref, brel_ref, gnw_ref, gnb_ref, gnms_ref,
                  wlt_ref, blin_ref, out_ref,
                  conv_ref, stats_ref, carry_ref, acc_ref):
    p = pl.program_id(0)
    b = pl.program_id(1)
    rows = pl.ds(b * _BLK, _BLK)
    pm1 = jnp.maximum(p - 1, 0)

    x0b = x0_ref[...]
    cin = conv_ref[rows, :]

    st = stats_ref[pm1]
    mean = st[0:1, :] * (1.0 / _N)
    ex2 = st[1:2, :] * (1.0 / _N)
    ms = gnms_ref[0]
    var = ex2 - ms * (2.0 - ms) * mean * mean
    inv = jax.lax.rsqrt(var + _EPS)
    y = gnw_ref[0] * (cin - ms * mean) * inv + gnb_ref[0]

    @pl.when(p <= 3)
    def _():
        xin = jnp.where(p == 0, x0b, jnp.where(y >= 0, y, _SLOPE * y))
        prod = jnp.dot(xin, w2_ref[0], preferred_element_type=jnp.float32)
        a = prod[:, :_D]
        bb = prod[:, _D:]

        @pl.when(b == 0)
        def _():
            carry_ref[...] = jnp.zeros_like(carry_ref)
            stats_ref[p] = jnp.zeros_like(stats_ref[p])

        conv = jnp.concatenate([carry_ref[...], a[:-1, :]], axis=0) \
            + bb + brel_ref[0]
        carry_ref[...] = a[-1:, :]
        conv_ref[rows, :] = conv
        stats_ref[p] += jnp.concatenate(
            [jnp.sum(conv, axis=0, keepdims=True),
             jnp.sum(conv * conv, axis=0, keepdims=True)], axis=0)

    @pl.when(p == 4)
    def _():
        z = jnp.maximum(x0b + y, 0.0)

        @pl.when(b == 0)
        def _():
            acc_ref[...] = jnp.zeros_like(acc_ref)

        acc_ref[...] += jnp.sum(z, axis=0, keepdims=True)

        @pl.when(b == _NB - 1)
        def _():
            pooled = acc_ref[...] * (1.0 / _N)
            logits = jnp.dot(pooled, wlt_ref[...],
                             preferred_element_type=jnp.float32) + blin_ref[...]
            out_ref[...] = jax.nn.softplus(logits)


def kernel(feat, W_rel0, b_rel0, W_root0, gn_w0, gn_b0, gn_ms0,
           W_rel1, b_rel1, W_root1, gn_w1, gn_b1, gn_ms1,
           W_rel2, b_rel2, W_root2, gn_w2, gn_b2, gn_ms2,
           W_rel3, b_rel3, W_root3, gn_w3, gn_b3, gn_ms3,
           W_lin, b_lin):
    x0 = feat[0]
    f32 = jnp.float32

    w2_all = jnp.stack([
        jnp.concatenate([Wr.T, Wo.T], axis=1)
        for Wr, Wo in ((W_rel0, W_root0), (W_rel1, W_root1),
                       (W_rel2, W_root2), (W_rel3, W_root3))])
    brel_all = jnp.stack([b_rel0, b_rel1, b_rel2, b_rel3])[:, None, :]
    gnw_all = jnp.stack([gn_w0, gn_w1, gn_w2, gn_w3])[:, None, :]
    gnb_all = jnp.stack([gn_b0, gn_b1, gn_b2, gn_b3])[:, None, :]
    gnms_all = jnp.stack([gn_ms0, gn_ms1, gn_ms2, gn_ms3])[:, None, :]

    def x0_map(p, b):
        return (jnp.where((p == 0) | (p == 4), b, 0), 0)

    def conv_w_map(p, b):
        return (jnp.minimum(p, 3), 0, 0)

    def gn_map(p, b):
        return (jnp.maximum(p - 1, 0), 0, 0)

    out = pl.pallas_call(
        _fused_kernel,
        grid=(5, _NB),
        in_specs=[
            pl.BlockSpec((_BLK, _D), x0_map),
            pl.BlockSpec((1, _D, 2 * _D), conv_w_map),
            pl.BlockSpec((1, 1, _D), conv_w_map),
            pl.BlockSpec((1, 1, _D), gn_map),
            pl.BlockSpec((1, 1, _D), gn_map),
            pl.BlockSpec((1, 1, _D), gn_map),
            pl.BlockSpec((_D, _C), lambda p, b: (0, 0)),
            pl.BlockSpec((1, _C), lambda p, b: (0, 0)),
        ],
        out_specs=pl.BlockSpec((1, _C), lambda p, b: (0, 0)),
        out_shape=jax.ShapeDtypeStruct((1, _C), f32),
        scratch_shapes=[
            pltpu.VMEM((_N, _D), f32),
            pltpu.VMEM((4, 2, _D), f32),
            pltpu.VMEM((1, _D), f32),
            pltpu.VMEM((1, _D), f32),
        ],
    )(x0, w2_all, brel_all, gnw_all, gnb_all, gnms_all, W_lin.T,
      b_lin.reshape(1, _C))

    return out.reshape(_C)


# speedup vs baseline: 14.6007x; 1.0051x over previous
import jax
import jax.numpy as jnp
from jax.experimental import pallas as pl
from jax.experimental.pallas import tpu as pltpu

_N = 100000
_D = 128
_C = 10
_BLK = 2000
_NB = _N // _BLK
_EPS = 1e-5
_SLOPE = 0.1


def _fused_kernel(x0_ref, w2_ref, brel_ref, gnw_ref, gnb_ref, gnms_ref,
                  wlt_ref, blin_ref, out_ref,
                  conv_ref, stats_ref, carry_ref, acc_ref):
    p = pl.program_id(0)
    b = pl.program_id(1)
    rows = pl.ds(b * _BLK, _BLK)
    pm1 = jnp.maximum(p - 1, 0)

    x0b = x0_ref[...]
    cin = conv_ref[rows, :].astype(jnp.float32)

    st = stats_ref[pm1]
    mean = st[0:1, :] * (1.0 / _N)
    ex2 = st[1:2, :] * (1.0 / _N)
    ms = gnms_ref[0]
    var = ex2 - ms * (2.0 - ms) * mean * mean
    inv = jax.lax.rsqrt(var + _EPS)
    y = gnw_ref[0] * (cin - ms * mean) * inv + gnb_ref[0]

    @pl.when(p <= 3)
    def _():
        xin = jnp.where(p == 0, x0b, jnp.where(y >= 0, y, _SLOPE * y))
        prod = jnp.dot(xin.astype(jnp.bfloat16),
                       w2_ref[0].astype(jnp.bfloat16),
                       preferred_element_type=jnp.float32)
        a = prod[:, :_D]
        bb = prod[:, _D:]

        @pl.when(b == 0)
        def _():
            carry_ref[...] = jnp.zeros_like(carry_ref)
            stats_ref[p] = jnp.zeros_like(stats_ref[p])

        conv = jnp.concatenate([carry_ref[...], a[:-1, :]], axis=0) \
            + bb + brel_ref[0]
        carry_ref[...] = a[-1:, :]
        conv_ref[rows, :] = conv.astype(jnp.bfloat16)
        stats_ref[p] += jnp.concatenate(
            [jnp.sum(conv, axis=0, keepdims=True),
             jnp.sum(conv * conv, axis=0, keepdims=True)], axis=0)

    @pl.when(p == 4)
    def _():
        z = jnp.maximum(x0b + y, 0.0)

        @pl.when(b == 0)
        def _():
            acc_ref[...] = jnp.zeros_like(acc_ref)

        acc_ref[...] += jnp.sum(z, axis=0, keepdims=True)

        @pl.when(b == _NB - 1)
        def _():
            pooled = acc_ref[...] * (1.0 / _N)
            logits = jnp.dot(pooled, wlt_ref[...],
                             preferred_element_type=jnp.float32) + blin_ref[...]
            out_ref[...] = jax.nn.softplus(logits)


def kernel(feat, W_rel0, b_rel0, W_root0, gn_w0, gn_b0, gn_ms0,
           W_rel1, b_rel1, W_root1, gn_w1, gn_b1, gn_ms1,
           W_rel2, b_rel2, W_root2, gn_w2, gn_b2, gn_ms2,
           W_rel3, b_rel3, W_root3, gn_w3, gn_b3, gn_ms3,
           W_lin, b_lin):
    x0 = feat[0]
    f32 = jnp.float32

    w2_all = jnp.stack([
        jnp.concatenate([Wr.T, Wo.T], axis=1)
        for Wr, Wo in ((W_rel0, W_root0), (W_rel1, W_root1),
                       (W_rel2, W_root2), (W_rel3, W_root3))])
    brel_all = jnp.stack([b_rel0, b_rel1, b_rel2, b_rel3])[:, None, :]
    gnw_all = jnp.stack([gn_w0, gn_w1, gn_w2, gn_w3])[:, None, :]
    gnb_all = jnp.stack([gn_b0, gn_b1, gn_b2, gn_b3])[:, None, :]
    gnms_all = jnp.stack([gn_ms0, gn_ms1, gn_ms2, gn_ms3])[:, None, :]

    def x0_map(p, b):
        return (jnp.where((p == 0) | (p == 4), b, 0), 0)

    def conv_w_map(p, b):
        return (jnp.minimum(p, 3), 0, 0)

    def gn_map(p, b):
        return (jnp.maximum(p - 1, 0), 0, 0)

    out = pl.pallas_call(
        _fused_kernel,
        grid=(5, _NB),
        in_specs=[
            pl.BlockSpec((_BLK, _D), x0_map),
            pl.BlockSpec((1, _D, 2 * _D), conv_w_map),
            pl.BlockSpec((1, 1, _D), conv_w_map),
            pl.BlockSpec((1, 1, _D), gn_map),
            pl.BlockSpec((1, 1, _D), gn_map),
            pl.BlockSpec((1, 1, _D), gn_map),
            pl.BlockSpec((_D, _C), lambda p, b: (0, 0)),
            pl.BlockSpec((1, _C), lambda p, b: (0, 0)),
        ],
        out_specs=pl.BlockSpec((1, _C), lambda p, b: (0, 0)),
        out_shape=jax.ShapeDtypeStruct((1, _C), f32),
        scratch_shapes=[
            pltpu.VMEM((_N, _D), jnp.bfloat16),
            pltpu.VMEM((4, 2, _D), f32),
            pltpu.VMEM((1, _D), f32),
            pltpu.VMEM((1, _D), f32),
        ],
    )(x0, w2_all, brel_all, gnw_all, gnb_all, gnms_all, W_lin.T,
      b_lin.reshape(1, _C))

    return out.reshape(_C)


# speedup vs baseline: 16.8593x; 1.1547x over previous
import jax
import jax.numpy as jnp
from jax.experimental import pallas as pl
from jax.experimental.pallas import tpu as pltpu

_N = 100000
_D = 128
_C = 10
_BLK = 5000
_NB = _N // _BLK
_EPS = 1e-5
_SLOPE = 0.1


def _fused_kernel(x0_ref, w2_ref, brel_ref, gnw_ref, gnb_ref, gnms_ref,
                  wlt_ref, blin_ref, out_ref,
                  conv_ref, stats_ref, carry_ref, acc_ref):
    p = pl.program_id(0)
    b = pl.program_id(1)
    rows = pl.ds(b * _BLK, _BLK)
    pm1 = jnp.maximum(p - 1, 0)

    x0b = x0_ref[...]
    cin = conv_ref[rows, :].astype(jnp.float32)

    st = stats_ref[pm1]
    mean = st[0:1, :] * (1.0 / _N)
    ex2 = st[1:2, :] * (1.0 / _N)
    ms = gnms_ref[0]
    var = ex2 - ms * (2.0 - ms) * mean * mean
    inv = jax.lax.rsqrt(var + _EPS)
    y = gnw_ref[0] * (cin - ms * mean) * inv + gnb_ref[0]

    @pl.when(p <= 3)
    def _():
        xin = jnp.where(p == 0, x0b, jnp.where(y >= 0, y, _SLOPE * y))
        prod = jnp.dot(xin.astype(jnp.bfloat16),
                       w2_ref[0].astype(jnp.bfloat16),
                       preferred_element_type=jnp.float32)
        a = prod[:, :_D]
        bb = prod[:, _D:]

        @pl.when(b == 0)
        def _():
            carry_ref[...] = jnp.zeros_like(carry_ref)
            stats_ref[p] = jnp.zeros_like(stats_ref[p])

        conv = jnp.concatenate([carry_ref[...], a[:-1, :]], axis=0) \
            + bb + brel_ref[0]
        carry_ref[...] = a[-1:, :]
        conv_ref[rows, :] = conv.astype(jnp.bfloat16)
        stats_ref[p] += jnp.concatenate(
            [jnp.sum(conv, axis=0, keepdims=True),
             jnp.sum(conv * conv, axis=0, keepdims=True)], axis=0)

    @pl.when(p == 4)
    def _():
        z = jnp.maximum(x0b + y, 0.0)

        @pl.when(b == 0)
        def _():
            acc_ref[...] = jnp.zeros_like(acc_ref)

        acc_ref[...] += jnp.sum(z, axis=0, keepdims=True)

        @pl.when(b == _NB - 1)
        def _():
            pooled = acc_ref[...] * (1.0 / _N)
            logits = jnp.dot(pooled, wlt_ref[...],
                             preferred_element_type=jnp.float32) + blin_ref[...]
            out_ref[...] = jax.nn.softplus(logits)


def kernel(feat, W_rel0, b_rel0, W_root0, gn_w0, gn_b0, gn_ms0,
           W_rel1, b_rel1, W_root1, gn_w1, gn_b1, gn_ms1,
           W_rel2, b_rel2, W_root2, gn_w2, gn_b2, gn_ms2,
           W_rel3, b_rel3, W_root3, gn_w3, gn_b3, gn_ms3,
           W_lin, b_lin):
    x0 = feat[0]
    f32 = jnp.float32

    w2_all = jnp.stack([
        jnp.concatenate([Wr.T, Wo.T], axis=1)
        for Wr, Wo in ((W_rel0, W_root0), (W_rel1, W_root1),
                       (W_rel2, W_root2), (W_rel3, W_root3))])
    brel_all = jnp.stack([b_rel0, b_rel1, b_rel2, b_rel3])[:, None, :]
    gnw_all = jnp.stack([gn_w0, gn_w1, gn_w2, gn_w3])[:, None, :]
    gnb_all = jnp.stack([gn_b0, gn_b1, gn_b2, gn_b3])[:, None, :]
    gnms_all = jnp.stack([gn_ms0, gn_ms1, gn_ms2, gn_ms3])[:, None, :]

    def x0_map(p, b):
        return (jnp.where((p == 0) | (p == 4), b, 0), 0)

    def conv_w_map(p, b):
        return (jnp.minimum(p, 3), 0, 0)

    def gn_map(p, b):
        return (jnp.maximum(p - 1, 0), 0, 0)

    out = pl.pallas_call(
        _fused_kernel,
        grid=(5, _NB),
        in_specs=[
            pl.BlockSpec((_BLK, _D), x0_map),
            pl.BlockSpec((1, _D, 2 * _D), conv_w_map),
            pl.BlockSpec((1, 1, _D), conv_w_map),
            pl.BlockSpec((1, 1, _D), gn_map),
            pl.BlockSpec((1, 1, _D), gn_map),
            pl.BlockSpec((1, 1, _D), gn_map),
            pl.BlockSpec((_D, _C), lambda p, b: (0, 0)),
            pl.BlockSpec((1, _C), lambda p, b: (0, 0)),
        ],
        out_specs=pl.BlockSpec((1, _C), lambda p, b: (0, 0)),
        out_shape=jax.ShapeDtypeStruct((1, _C), f32),
        scratch_shapes=[
            pltpu.VMEM((_N, _D), jnp.bfloat16),
            pltpu.VMEM((4, 2, _D), f32),
            pltpu.VMEM((1, _D), f32),
            pltpu.VMEM((1, _D), f32),
        ],
    )(x0, w2_all, brel_all, gnw_all, gnb_all, gnms_all, W_lin.T,
      b_lin.reshape(1, _C))

    return out.reshape(_C)


# speedup vs baseline: 19.3929x; 1.1503x over previous
import jax
import jax.numpy as jnp
from jax.experimental import pallas as pl
from jax.experimental.pallas import tpu as pltpu

_N = 100000
_D = 128
_C = 10
_BLK = 10000
_NB = _N // _BLK
_EPS = 1e-5
_SLOPE = 0.1


def _fused_kernel(x0b_ref, x0f_ref, ones_ref, w2_ref, brel_ref,
                  gnw_ref, gnb_ref, gnms_ref, wlt_ref, blin_ref, out_ref,
                  conv_ref, s1_ref, s2_ref, carry_ref, acc_ref):
    p = pl.program_id(0)
    b = pl.program_id(1)
    rows = pl.ds(b * _BLK, _BLK)
    pm1 = jnp.maximum(p - 1, 0)
    bf16 = jnp.bfloat16

    def norm_coeffs():
        mean = s1_ref[pm1, 0:1, :] * (1.0 / _N)
        ex2 = s2_ref[pm1, 0:1, :] * (1.0 / _N)
        ms = gnms_ref[0]
        var = ex2 - ms * (2.0 - ms) * mean * mean
        alpha = gnw_ref[0] * jax.lax.rsqrt(var + _EPS)
        beta = gnb_ref[0] - alpha * ms * mean
        return alpha, beta

    def conv_tail(xin):
        prod = jnp.dot(xin, w2_ref[0], preferred_element_type=jnp.float32)
        a = prod[:, :_D]
        bb = prod[:, _D:]

        @pl.when(b == 0)
        def _():
            carry_ref[...] = jnp.zeros_like(carry_ref)
            s1_ref[p] = jnp.zeros_like(s1_ref[p])
            s2_ref[p] = jnp.zeros_like(s2_ref[p])

        conv = jnp.concatenate([carry_ref[...], a[:-1, :]], axis=0) \
            + bb + brel_ref[0]
        carry_ref[...] = a[-1:, :]
        cb = conv.astype(bf16)
        conv_ref[rows, :] = cb
        dims = (((0,), (0,)), ((), ()))
        s1_ref[p] += jax.lax.dot_general(
            ones_ref[...], cb, dims, preferred_element_type=jnp.float32)
        s2_ref[p] += jax.lax.dot_general(
            ones_ref[...], cb * cb, dims,
            preferred_element_type=jnp.float32)

    @pl.when(p == 0)
    def _():
        conv_tail(x0b_ref[...])

    @pl.when((p >= 1) & (p <= 3))
    def _():
        alpha, beta = norm_coeffs()
        y = conv_ref[rows, :] * alpha.astype(bf16) + beta.astype(bf16)
        conv_tail(jnp.maximum(y, bf16(_SLOPE) * y))

    @pl.when(p == 4)
    def _():
        alpha, beta = norm_coeffs()
        y = conv_ref[rows, :].astype(jnp.float32) * alpha + beta
        z = jnp.maximum(x0f_ref[...] + y, 0.0)

        @pl.when(b == 0)
        def _():
            acc_ref[...] = jnp.zeros_like(acc_ref)

        acc_ref[...] += jnp.sum(z, axis=0, keepdims=True)

        @pl.when(b == _NB - 1)
        def _():
            pooled = acc_ref[...] * (1.0 / _N)
            logits = jnp.dot(pooled, wlt_ref[...],
                             preferred_element_type=jnp.float32) + blin_ref[...]
            out_ref[...] = jax.nn.softplus(logits)


def kernel(feat, W_rel0, b_rel0, W_root0, gn_w0, gn_b0, gn_ms0,
           W_rel1, b_rel1, W_root1, gn_w1, gn_b1, gn_ms1,
           W_rel2, b_rel2, W_root2, gn_w2, gn_b2, gn_ms2,
           W_rel3, b_rel3, W_root3, gn_w3, gn_b3, gn_ms3,
           W_lin, b_lin):
    x0 = feat[0]
    f32 = jnp.float32
    bf16 = jnp.bfloat16

    w2_all = jnp.stack([
        jnp.concatenate([Wr.T, Wo.T], axis=1)
        for Wr, Wo in ((W_rel0, W_root0), (W_rel1, W_root1),
                       (W_rel2, W_root2), (W_rel3, W_root3))]).astype(bf16)
    brel_all = jnp.stack([b_rel0, b_rel1, b_rel2, b_rel3])[:, None, :]
    gnw_all = jnp.stack([gn_w0, gn_w1, gn_w2, gn_w3])[:, None, :]
    gnb_all = jnp.stack([gn_b0, gn_b1, gn_b2, gn_b3])[:, None, :]
    gnms_all = jnp.stack([gn_ms0, gn_ms1, gn_ms2, gn_ms3])[:, None, :]
    ones_col = jnp.ones((_BLK, 8), bf16)

    def x0b_map(p, b):
        return (jnp.where(p == 0, b, 0), 0)

    def x0f_map(p, b):
        return (jnp.where(p == 4, b, 0), 0)

    def conv_w_map(p, b):
        return (jnp.minimum(p, 3), 0, 0)

    def gn_map(p, b):
        return (jnp.maximum(p - 1, 0), 0, 0)

    out = pl.pallas_call(
        _fused_kernel,
        grid=(5, _NB),
        in_specs=[
            pl.BlockSpec((_BLK, _D), x0b_map),
            pl.BlockSpec((_BLK, _D), x0f_map),
            pl.BlockSpec((_BLK, 8), lambda p, b: (0, 0)),
            pl.BlockSpec((1, _D, 2 * _D), conv_w_map),
            pl.BlockSpec((1, 1, _D), conv_w_map),
            pl.BlockSpec((1, 1, _D), gn_map),
            pl.BlockSpec((1, 1, _D), gn_map),
            pl.BlockSpec((1, 1, _D), gn_map),
            pl.BlockSpec((_D, _C), lambda p, b: (0, 0)),
            pl.BlockSpec((1, _C), lambda p, b: (0, 0)),
        ],
        out_specs=pl.BlockSpec((1, _C), lambda p, b: (0, 0)),
        out_shape=jax.ShapeDtypeStruct((1, _C), f32),
        scratch_shapes=[
            pltpu.VMEM((_N, _D), bf16),
            pltpu.VMEM((4, 8, _D), f32),
            pltpu.VMEM((4, 8, _D), f32),
            pltpu.VMEM((1, _D), f32),
            pltpu.VMEM((1, _D), f32),
        ],
    )(x0.astype(bf16), x0, ones_col, w2_all, brel_all, gnw_all, gnb_all,
      gnms_all, W_lin.T, b_lin.reshape(1, _C))

    return out.reshape(_C)
